# Initial kernel scaffold; baseline (speedup 1.0000x reference)
import jax
import jax.numpy as jnp
from jax.experimental import pallas as pl


def kernel(features, neighborhood):
    raise NotImplementedError("write your pallas kernel here")


# speedup vs baseline: 5.1592x; 5.1592x over previous
import functools

import jax
import jax.numpy as jnp
from jax import lax
from jax.experimental import pallas as pl
from jax.experimental.pallas import tpu as pltpu
from jax.experimental.pallas import tpu_sc as plsc

D = 128
N = 10000
K = 32
NW = 32
PTS_PER_W = 320
NPAD = NW * PTS_PER_W
GROUP_IDX = 128
PTS_PER_GROUP = GROUP_IDX // K
NGROUPS = PTS_PER_W // PTS_PER_GROUP
NLANE = 16
NCOL = D // NLANE


def _mesh():
    return plsc.VectorSubcoreMesh(core_axis_name="c", subcore_axis_name="s")


@functools.partial(
    pl.kernel,
    out_type=jax.ShapeDtypeStruct((NPAD, D), jnp.float32),
    mesh=_mesh(),
    scratch_types=[
        pltpu.VMEM((NGROUPS, GROUP_IDX), jnp.int32),
        pltpu.VMEM((GROUP_IDX, D), jnp.float32),
        pltpu.VMEM((PTS_PER_W, D), jnp.float32),
        pltpu.SemaphoreType.DMA,
    ],
)
def _sc_flex_max_pool(fT_hbm, idx_hbm, out_hbm, idx_v, rows_v, out_v, sem):
    wid = lax.axis_index("s") * 2 + lax.axis_index("c")
    pltpu.sync_copy(idx_hbm.at[wid], idx_v)

    def compute_group(j, buf):
        for p in range(PTS_PER_GROUP):
            base = K * p

            def rbody(r, accs):
                return tuple(
                    jnp.maximum(accs[c], buf[base + r, pl.ds(NLANE * c, NLANE)])
                    for c in range(NCOL)
                )

            accs = tuple(
                buf[base, pl.ds(NLANE * c, NLANE)] for c in range(NCOL)
            )
            accs = lax.fori_loop(1, K, rbody, accs, unroll=4)
            row = PTS_PER_GROUP * j + p
            for c in range(NCOL):
                out_v[row, pl.ds(NLANE * c, NLANE)] = accs[c]

    def gbody(j, carry):
        cp = pltpu.make_async_copy(fT_hbm.at[idx_v.at[j]], rows_v, sem)
        cp.start()
        cp.wait()
        compute_group(j, rows_v)
        return carry

    lax.fori_loop(0, NGROUPS, gbody, 0)
    pltpu.sync_copy(out_v, out_hbm.at[pl.ds(wid * PTS_PER_W, PTS_PER_W)])


def kernel(features, neighborhood):
    fT = features[0].T
    nbT = neighborhood[0].T
    nbp = jnp.pad(nbT, ((0, NPAD - N), (0, 0)))
    idx = nbp.reshape(NW, NGROUPS, GROUP_IDX)
    outT = _sc_flex_max_pool(fT, idx)
    return outT[:N].T[None]


# speedup vs baseline: 5.6537x; 1.0958x over previous
import functools

import jax
import jax.numpy as jnp
from jax import lax
from jax.experimental import pallas as pl
from jax.experimental.pallas import tpu as pltpu
from jax.experimental.pallas import tpu_sc as plsc

D = 128
N = 10000
K = 32
NW = 32
PTS_PER_W = 320
NPAD = NW * PTS_PER_W
GROUP_IDX = 128
PTS_PER_GROUP = GROUP_IDX // K
NGROUPS = PTS_PER_W // PTS_PER_GROUP
NLANE = 16
NCOL = D // NLANE


def _mesh():
    return plsc.VectorSubcoreMesh(core_axis_name="c", subcore_axis_name="s")


@functools.partial(
    pl.kernel,
    out_type=jax.ShapeDtypeStruct((NPAD, D), jnp.float32),
    mesh=_mesh(),
    scratch_types=[
        pltpu.VMEM((NGROUPS, GROUP_IDX), jnp.int32),
        pltpu.VMEM((GROUP_IDX, D), jnp.float32),
        pltpu.VMEM((GROUP_IDX, D), jnp.float32),
        pltpu.VMEM((PTS_PER_W, D), jnp.float32),
        pltpu.SemaphoreType.DMA,
        pltpu.SemaphoreType.DMA,
    ],
)
def _sc_flex_max_pool(fT_hbm, idx_hbm, out_hbm, idx_v, buf0, buf1, out_v,
                      sem0, sem1):
    wid = lax.axis_index("s") * 2 + lax.axis_index("c")
    pltpu.sync_copy(idx_hbm.at[wid], idx_v)

    def start(j, buf, sem):
        pltpu.make_async_copy(fT_hbm.at[idx_v.at[j]], buf, sem).start()

    def wait(j, buf, sem):
        pltpu.make_async_copy(fT_hbm.at[idx_v.at[j]], buf, sem).wait()

    def compute_group(j, buf):
        for p in range(PTS_PER_GROUP):
            base = K * p

            def rbody(r, accs):
                return tuple(
                    jnp.maximum(accs[c], buf[base + r, pl.ds(NLANE * c, NLANE)])
                    for c in range(NCOL)
                )

            accs = tuple(
                buf[base, pl.ds(NLANE * c, NLANE)] for c in range(NCOL)
            )
            accs = lax.fori_loop(1, K, rbody, accs, unroll=4)
            row = PTS_PER_GROUP * j + p
            for c in range(NCOL):
                out_v[row, pl.ds(NLANE * c, NLANE)] = accs[c]

    start(0, buf0, sem0)

    def gbody(jj, carry):
        g0 = 2 * jj
        start(g0 + 1, buf1, sem1)
        wait(g0, buf0, sem0)
        compute_group(g0, buf0)
        start(g0 + 2, buf0, sem0)
        wait(g0 + 1, buf1, sem1)
        compute_group(g0 + 1, buf1)
        return carry

    lax.fori_loop(0, NGROUPS // 2 - 1, gbody, 0)
    gl = NGROUPS - 2
    start(gl + 1, buf1, sem1)
    wait(gl, buf0, sem0)
    compute_group(gl, buf0)
    wait(gl + 1, buf1, sem1)
    compute_group(gl + 1, buf1)
    pltpu.sync_copy(out_v, out_hbm.at[pl.ds(wid * PTS_PER_W, PTS_PER_W)])


def kernel(features, neighborhood):
    fT = features[0].T
    nbT = neighborhood[0].T
    nbp = jnp.pad(nbT, ((0, NPAD - N), (0, 0)))
    idx = nbp.reshape(NW, NGROUPS, GROUP_IDX)
    outT = _sc_flex_max_pool(fT, idx)
    return outT[:N].T[None]


# speedup vs baseline: 11.5935x; 2.0506x over previous
import functools

import jax
import jax.numpy as jnp
from jax import lax
from jax.experimental import pallas as pl
from jax.experimental.pallas import tpu as pltpu
from jax.experimental.pallas import tpu_sc as plsc

D = 128
N = 10000
K = 32
NC = 2
NS = 16
PTS_PER_T = 640
NPAD = NS * PTS_PER_T
HALF = NPAD // 2
SENT = HALF
TPAD = HALF + 128
ROWS_PER_T = TPAD // NS
GROUP_IDX = 128
PTS_PER_GROUP = GROUP_IDX // K
NGROUPS = PTS_PER_T // PTS_PER_GROUP
NPHASES = 4
FLUSH_GROUPS = NGROUPS // NPHASES
FLUSH_PTS = FLUSH_GROUPS * PTS_PER_GROUP
NLANE = 16
NCOL = D // NLANE


def _mesh():
    return plsc.VectorSubcoreMesh(core_axis_name="c", subcore_axis_name="s")


@functools.partial(
    pl.kernel,
    out_type=jax.ShapeDtypeStruct((NC, NPAD, D), jnp.float32),
    mesh=_mesh(),
    scratch_types=[
        pltpu.VMEM_SHARED((TPAD, D), jnp.float32),
        pltpu.VMEM((NGROUPS, GROUP_IDX), jnp.int32),
        pltpu.VMEM((GROUP_IDX,), jnp.int32),
        pltpu.VMEM((GROUP_IDX,), jnp.int32),
        pltpu.VMEM((GROUP_IDX, D), jnp.float32),
        pltpu.VMEM((GROUP_IDX, D), jnp.float32),
        pltpu.VMEM((FLUSH_PTS, D), jnp.float32),
        pltpu.SemaphoreType.DMA,
        pltpu.SemaphoreType.DMA,
    ],
)
def _sc_flex_max_pool(tbl_hbm, idx_hbm, out_hbm, f_sp, idx_v, idxt0, idxt1,
                      buf0, buf1, out_v, sem0, sem1):
    cid = lax.axis_index("c")
    sid = lax.axis_index("s")
    cbase = cid * HALF

    rbase = sid * ROWS_PER_T
    off = 0
    for nrows in (112, 112, 104):
        pltpu.sync_copy(tbl_hbm.at[cid, pl.ds(rbase + off, nrows)],
                        buf0.at[pl.ds(0, nrows)])
        pltpu.sync_copy(buf0.at[pl.ds(0, nrows)],
                        f_sp.at[pl.ds(rbase + off, nrows)])
        off += nrows
    pltpu.sync_copy(idx_hbm.at[sid], idx_v)
    plsc.subcore_barrier()

    def remap(j, idxt):
        for c in range(GROUP_IDX // NLANE):
            v = idx_v[j, pl.ds(NLANE * c, NLANE)]
            loc = v - cbase
            ok = (loc >= 0) & (loc < HALF)
            idxt[pl.ds(NLANE * c, NLANE)] = jnp.where(ok, loc, SENT)

    def start(j, idxt, buf, sem):
        remap(j, idxt)
        pltpu.make_async_copy(f_sp.at[idxt], buf, sem).start()

    def wait(idxt, buf, sem):
        pltpu.make_async_copy(f_sp.at[idxt], buf, sem).wait()

    def compute_group(j, jloc, buf):
        for p in range(PTS_PER_GROUP):
            base = K * p

            def rbody(r, accs):
                return tuple(
                    jnp.maximum(accs[c], buf[base + r, pl.ds(NLANE * c, NLANE)])
                    for c in range(NCOL)
                )

            accs = tuple(
                buf[base, pl.ds(NLANE * c, NLANE)] for c in range(NCOL)
            )
            accs = lax.fori_loop(1, K, rbody, accs, unroll=4)
            row = PTS_PER_GROUP * jloc + p
            for c in range(NCOL):
                out_v[row, pl.ds(NLANE * c, NLANE)] = accs[c]

    for ph in range(NPHASES):
        g = ph * FLUSH_GROUPS
        start(g, idxt0, buf0, sem0)

        def gbody(jj, carry, g=g):
            g0 = g + 2 * jj
            start(g0 + 1, idxt1, buf1, sem1)
            wait(idxt0, buf0, sem0)
            compute_group(g0, g0 - g, buf0)
            start(g0 + 2, idxt0, buf0, sem0)
            wait(idxt1, buf1, sem1)
            compute_group(g0 + 1, g0 + 1 - g, buf1)
            return carry

        lax.fori_loop(0, FLUSH_GROUPS // 2 - 1, gbody, 0)
        gl = g + FLUSH_GROUPS - 2
        start(gl + 1, idxt1, buf1, sem1)
        wait(idxt0, buf0, sem0)
        compute_group(gl, gl - g, buf0)
        wait(idxt1, buf1, sem1)
        compute_group(gl + 1, gl + 1 - g, buf1)
        pltpu.sync_copy(
            out_v,
            out_hbm.at[cid, pl.ds(sid * PTS_PER_T + ph * FLUSH_PTS,
                                  FLUSH_PTS)])


def kernel(features, neighborhood):
    fT = features[0].T
    fT = jnp.pad(fT, ((0, NPAD - N), (0, 0)))
    neg = jnp.full((TPAD - HALF, D), -jnp.inf, jnp.float32)
    tbl = jnp.stack([jnp.concatenate([fT[:HALF], neg]),
                     jnp.concatenate([fT[HALF:], neg])])
    nbT = neighborhood[0].T
    nbp = jnp.pad(nbT, ((0, NPAD - N), (0, 0)))
    idx = nbp.reshape(NS, NGROUPS, GROUP_IDX)
    parts = _sc_flex_max_pool(tbl, idx)
    outT = jnp.maximum(parts[0], parts[1])[:N]
    return outT.T[None]


# speedup vs baseline: 16.0917x; 1.3880x over previous
import functools

import jax
import jax.numpy as jnp
from jax import lax
from jax.experimental import pallas as pl
from jax.experimental.pallas import tpu as pltpu
from jax.experimental.pallas import tpu_sc as plsc

D = 128
N = 10000
K = 32
NC = 2
NS = 16
PTS_PER_T = 640
NPAD = NS * PTS_PER_T
HALF = NPAD // 2
SENT = HALF
SENT_ROWS = 128
TPAD = HALF + SENT_ROWS
ROWS_PER_T = TPAD // NS
GROUP_IDX = 128
PTS_PER_GROUP = GROUP_IDX // K
NGROUPS = PTS_PER_T // PTS_PER_GROUP
NPHASES = 4
FLUSH_GROUPS = NGROUPS // NPHASES
FLUSH_PTS = FLUSH_GROUPS * PTS_PER_GROUP
NLANE = 16
NCOL = D // NLANE


def _mesh():
    return plsc.VectorSubcoreMesh(core_axis_name="c", subcore_axis_name="s")


@functools.partial(
    pl.kernel,
    out_type=jax.ShapeDtypeStruct((NC, NPAD, D), jnp.float32),
    mesh=_mesh(),
    scratch_types=[
        pltpu.VMEM_SHARED((TPAD, D), jnp.float32),
        pltpu.VMEM((NGROUPS, GROUP_IDX), jnp.int32),
        [pltpu.VMEM((GROUP_IDX,), jnp.int32)] * 2,
        [pltpu.VMEM((GROUP_IDX, D), jnp.float32)] * 2,
        pltpu.VMEM((FLUSH_PTS, D), jnp.float32),
        [pltpu.SemaphoreType.DMA] * 2,
    ],
)
def _sc_flex_max_pool(tbl_hbm, idx_hbm, out_hbm, f_sp, idx_v, idxts,
                      bufs, out_v, sems):
    cid = lax.axis_index("c")
    sid = lax.axis_index("s")
    cbase = cid * HALF

    rbase = sid * ROWS_PER_T
    off = 0
    for nrows in (112, 112, 104):
        pltpu.sync_copy(tbl_hbm.at[cid, pl.ds(rbase + off, nrows)],
                        bufs[0].at[pl.ds(0, nrows)])
        pltpu.sync_copy(bufs[0].at[pl.ds(0, nrows)],
                        f_sp.at[pl.ds(rbase + off, nrows)])
        off += nrows
    pltpu.sync_copy(idx_hbm.at[sid], idx_v)
    plsc.subcore_barrier()

    lanes = lax.iota(jnp.int32, NLANE)

    def remap(j, idxt):
        for c in range(GROUP_IDX // NLANE):
            v = idx_v[j, pl.ds(NLANE * c, NLANE)]
            loc = v - cbase
            ok = (loc >= 0) & (loc < HALF)
            sent = (SENT + NLANE * (c % (SENT_ROWS // NLANE))) + lanes
            idxt[pl.ds(NLANE * c, NLANE)] = jnp.where(ok, loc, sent)

    def start(j, idxt, buf, sem):
        remap(j, idxt)
        pltpu.make_async_copy(f_sp.at[idxt], buf, sem).start()

    def wait(idxt, buf, sem):
        pltpu.make_async_copy(f_sp.at[idxt], buf, sem).wait()

    def compute_group(jloc, buf):
        for p in range(PTS_PER_GROUP):
            base = K * p

            def rbody(r, accs):
                return tuple(
                    jnp.maximum(accs[c], buf[base + r, pl.ds(NLANE * c, NLANE)])
                    for c in range(NCOL)
                )

            accs = tuple(
                buf[base, pl.ds(NLANE * c, NLANE)] for c in range(NCOL)
            )
            accs = lax.fori_loop(1, K, rbody, accs, unroll=4)
            row = PTS_PER_GROUP * jloc + p
            for c in range(NCOL):
                out_v[row, pl.ds(NLANE * c, NLANE)] = accs[c]

    for ph in range(NPHASES):
        g = ph * FLUSH_GROUPS
        start(g, idxts[0], bufs[0], sems[0])

        def gbody(jj, carry, g=g):
            jloc = 2 * jj
            start(g + jloc + 1, idxts[1], bufs[1], sems[1])
            wait(idxts[0], bufs[0], sems[0])
            compute_group(jloc, bufs[0])
            start(g + jloc + 2, idxts[0], bufs[0], sems[0])
            wait(idxts[1], bufs[1], sems[1])
            compute_group(jloc + 1, bufs[1])
            return carry

        lax.fori_loop(0, FLUSH_GROUPS // 2 - 1, gbody, 0)
        gl = FLUSH_GROUPS - 2
        start(g + gl + 1, idxts[1], bufs[1], sems[1])
        wait(idxts[0], bufs[0], sems[0])
        compute_group(gl, bufs[0])
        wait(idxts[1], bufs[1], sems[1])
        compute_group(gl + 1, bufs[1])
        pltpu.sync_copy(
            out_v,
            out_hbm.at[cid, pl.ds(sid * PTS_PER_T + ph * FLUSH_PTS,
                                  FLUSH_PTS)])


def kernel(features, neighborhood):
    fT = features[0].T
    fT = jnp.pad(fT, ((0, NPAD - N), (0, 0)))
    neg = jnp.full((TPAD - HALF, D), -jnp.inf, jnp.float32)
    tbl = jnp.stack([jnp.concatenate([fT[:HALF], neg]),
                     jnp.concatenate([fT[HALF:], neg])])
    nbT = neighborhood[0].T
    nbp = jnp.pad(nbT, ((0, NPAD - N), (0, 0)))
    idx = nbp.reshape(NS, NGROUPS, GROUP_IDX)
    parts = _sc_flex_max_pool(tbl, idx)
    outT = jnp.maximum(parts[0], parts[1])[:N]
    return outT.T[None]


# speedup vs baseline: 16.6927x; 1.0374x over previous
import functools

import jax
import jax.numpy as jnp
from jax import lax
from jax.experimental import pallas as pl
from jax.experimental.pallas import tpu as pltpu
from jax.experimental.pallas import tpu_sc as plsc

D = 128
N = 10000
K = 32
NC = 2
NS = 16
PTS_PER_T = 640
NPAD = NS * PTS_PER_T
HALF = NPAD // 2
SENT = HALF
SENT_ROWS = 128
TPAD = HALF + SENT_ROWS
ROWS_PER_T = TPAD // NS
GROUP_IDX = 128
PTS_PER_GROUP = GROUP_IDX // K
NGROUPS = PTS_PER_T // PTS_PER_GROUP
NPHASES = 4
FLUSH_GROUPS = NGROUPS // NPHASES
FLUSH_PTS = FLUSH_GROUPS * PTS_PER_GROUP
NLANE = 16
NCOL = D // NLANE


def _mesh():
    return plsc.VectorSubcoreMesh(core_axis_name="c", subcore_axis_name="s")


@functools.partial(
    pl.kernel,
    out_type=jax.ShapeDtypeStruct((NC, NPAD, D), jnp.float32),
    mesh=_mesh(),
    scratch_types=[
        pltpu.VMEM_SHARED((TPAD, D), jnp.float32),
        pltpu.VMEM((NGROUPS, GROUP_IDX), jnp.int32),
        [pltpu.VMEM((GROUP_IDX,), jnp.int32)] * 2,
        [pltpu.VMEM((GROUP_IDX, D), jnp.float32)] * 2,
        pltpu.VMEM((FLUSH_PTS, D), jnp.float32),
        [pltpu.SemaphoreType.DMA] * 2,
    ],
)
def _sc_flex_max_pool(tbl_hbm, idx_hbm, out_hbm, f_sp, idx_v, idxts,
                      bufs, out_v, sems):
    cid = lax.axis_index("c")
    sid = lax.axis_index("s")
    cbase = cid * HALF

    rbase = sid * ROWS_PER_T
    off = 0
    for nrows in (112, 112, 104):
        pltpu.sync_copy(tbl_hbm.at[cid, pl.ds(rbase + off, nrows)],
                        bufs[0].at[pl.ds(0, nrows)])
        pltpu.sync_copy(bufs[0].at[pl.ds(0, nrows)],
                        f_sp.at[pl.ds(rbase + off, nrows)])
        off += nrows
    pltpu.sync_copy(idx_hbm.at[sid], idx_v)
    plsc.subcore_barrier()

    lanes = lax.iota(jnp.int32, NLANE)

    def remap(j, idxt):
        for c in range(GROUP_IDX // NLANE):
            v = idx_v[j, pl.ds(NLANE * c, NLANE)]
            loc = v - cbase
            ok = (loc >= 0) & (loc < HALF)
            sent = (SENT + NLANE * (c % (SENT_ROWS // NLANE))) + lanes
            idxt[pl.ds(NLANE * c, NLANE)] = jnp.where(ok, loc, sent)

    def start(j, idxt, buf, sem):
        remap(j, idxt)
        pltpu.make_async_copy(f_sp.at[idxt], buf, sem).start()

    def wait(idxt, buf, sem):
        pltpu.make_async_copy(f_sp.at[idxt], buf, sem).wait()

    def compute_group(jloc, buf):
        for p in range(PTS_PER_GROUP):
            base = K * p

            def rbody(r, accs):
                return tuple(
                    jnp.maximum(accs[c], buf[base + r, pl.ds(NLANE * c, NLANE)])
                    for c in range(NCOL)
                )

            accs = tuple(
                buf[base, pl.ds(NLANE * c, NLANE)] for c in range(NCOL)
            )
            accs = lax.fori_loop(1, K, rbody, accs, unroll=4)
            row = PTS_PER_GROUP * jloc + p
            for c in range(NCOL):
                out_v[row, pl.ds(NLANE * c, NLANE)] = accs[c]

    for ph in range(NPHASES):
        g = ph * FLUSH_GROUPS
        start(g, idxts[0], bufs[0], sems[0])

        def gbody(jj, carry, g=g):
            jloc = 2 * jj
            start(g + jloc + 1, idxts[1], bufs[1], sems[1])
            wait(idxts[0], bufs[0], sems[0])
            compute_group(jloc, bufs[0])
            start(g + jloc + 2, idxts[0], bufs[0], sems[0])
            wait(idxts[1], bufs[1], sems[1])
            compute_group(jloc + 1, bufs[1])
            return carry

        lax.fori_loop(0, FLUSH_GROUPS // 2 - 1, gbody, 0)
        gl = FLUSH_GROUPS - 2
        start(g + gl + 1, idxts[1], bufs[1], sems[1])
        wait(idxts[0], bufs[0], sems[0])
        compute_group(gl, bufs[0])
        wait(idxts[1], bufs[1], sems[1])
        compute_group(gl + 1, bufs[1])
        pltpu.sync_copy(
            out_v,
            out_hbm.at[cid, pl.ds(sid * PTS_PER_T + ph * FLUSH_PTS,
                                  FLUSH_PTS)])


def kernel(features, neighborhood):
    fT = features[0].T
    fT = jnp.pad(fT, ((0, NPAD - N), (0, 0)))
    tbl = jnp.pad(fT.reshape(NC, HALF, D),
                  ((0, 0), (0, TPAD - HALF), (0, 0)),
                  constant_values=-jnp.inf)
    nbT = neighborhood[0].T
    nbp = jnp.pad(nbT, ((0, NPAD - N), (0, 0)))
    idx = nbp.reshape(NS, NGROUPS, GROUP_IDX)
    parts = _sc_flex_max_pool(tbl, idx)
    outT = jnp.maximum(parts[0], parts[1])[:N]
    return outT.T[None]
